# baseline (speedup 1.0000x reference)
import jax
import jax.numpy as jnp
from jax import lax
from jax.experimental import pallas as pl
from jax.experimental.pallas import tpu as pltpu
from jax.experimental.pallas import tpu_sc as plsc

N_CORES = 2
N_SUBCORES = 16
NW = N_CORES * N_SUBCORES
LANES = 16
BATCH = 16384
K = 32
BPW = BATCH // NW
CHUNK = 128
NCH = BPW // CHUNK
GPC = CHUNK // LANES


def _mf_body(uid_hbm, iid_hbm, user_hbm, item_hbm, bu_hbm, bi_hbm, bias_hbm,
             out_hbm,
             idx_u, idx_i, ubuf0, ubuf1, ibuf0, ibuf1, bu_v, bi_v, bias_v,
             out_v, sem):
    cid = lax.axis_index("c")
    sid = lax.axis_index("s")
    wid = sid * N_CORES + cid
    base = wid * BPW

    for j in range(NCH):
        pltpu.sync_copy(uid_hbm.at[pl.ds(base + j * CHUNK, CHUNK)], idx_u.at[j])
        pltpu.sync_copy(iid_hbm.at[pl.ds(base + j * CHUNK, CHUNK)], idx_i.at[j])
    pltpu.sync_copy(bu_hbm.at[pl.ds(base, BPW)], bu_v)
    pltpu.sync_copy(bi_hbm.at[pl.ds(base, BPW)], bi_v)
    pltpu.sync_copy(bias_hbm, bias_v)

    bvec = bias_v[...]
    iota16 = lax.iota(jnp.int32, LANES)

    ubufs = (ubuf0, ubuf1)
    ibufs = (ibuf0, ibuf1)

    def fire(j):
        return (pltpu.async_copy(user_hbm.at[idx_u.at[j]], ubufs[j % 2], sem),
                pltpu.async_copy(item_hbm.at[idx_i.at[j]], ibufs[j % 2], sem))

    def compute(j):
        ub = ubufs[j % 2]
        ib = ibufs[j % 2]

        def group_body(g, carry):
            outv = jnp.zeros((LANES,), jnp.float32)
            for l in range(LANES):
                r = g * LANES + l
                u0 = ub[r, pl.ds(0, LANES)]
                u1 = ub[r, pl.ds(LANES, LANES)]
                i0 = ib[r, pl.ds(0, LANES)]
                i1 = ib[r, pl.ds(LANES, LANES)]
                prod = u0 * i0 + u1 * i1
                outv = jnp.where(iota16 == l, jnp.sum(prod), outv)
            o = j * CHUNK + g * LANES
            bu16 = bu_v[pl.ds(o, LANES)]
            bi16 = bi_v[pl.ds(o, LANES)]
            out_v[pl.ds(o, LANES)] = outv + bu16 + bi16 + bvec
            return carry

        lax.fori_loop(0, GPC, group_body, 0)

    pending = {0: fire(0), 1: fire(1)}
    for j in range(NCH):
        for c in pending.pop(j):
            c.wait()
        compute(j)
        if j + 2 < NCH:
            pending[j + 2] = fire(j + 2)

    pltpu.sync_copy(out_v, out_hbm.at[pl.ds(base, BPW)])


_mf = pl.kernel(
    _mf_body,
    mesh=plsc.VectorSubcoreMesh(core_axis_name="c", subcore_axis_name="s"),
    out_type=jax.ShapeDtypeStruct((BATCH,), jnp.float32),
    compiler_params=pltpu.CompilerParams(needs_layout_passes=False,
                                         use_tc_tiling_on_sc=False),
    scratch_types=[
        pltpu.VMEM((NCH, CHUNK), jnp.int32),
        pltpu.VMEM((NCH, CHUNK), jnp.int32),
        pltpu.VMEM((CHUNK, K), jnp.float32),
        pltpu.VMEM((CHUNK, K), jnp.float32),
        pltpu.VMEM((CHUNK, K), jnp.float32),
        pltpu.VMEM((CHUNK, K), jnp.float32),
        pltpu.VMEM((BPW,), jnp.float32),
        pltpu.VMEM((BPW,), jnp.float32),
        pltpu.VMEM((LANES,), jnp.float32),
        pltpu.VMEM((BPW,), jnp.float32),
        pltpu.SemaphoreType.DMA,
    ],
)


def kernel(train_x, user_w, item_w, bias_user_w, bias_item_w, bias):
    uid = train_x[:, 0]
    iid = train_x[:, 1]
    bu = jnp.take(bias_user_w.reshape(-1), uid, mode="clip")
    bi = jnp.take(bias_item_w.reshape(-1), iid, mode="clip")
    bias16 = jnp.broadcast_to(bias, (LANES,))
    return _mf(uid, iid, user_w, item_w, bu, bi, bias16)


# speedup vs baseline: 1.3039x; 1.3039x over previous
import jax
import jax.numpy as jnp
from jax import lax
from jax.experimental import pallas as pl
from jax.experimental.pallas import tpu as pltpu
from jax.experimental.pallas import tpu_sc as plsc

N_CORES = 2
N_SUBCORES = 16
NW = N_CORES * N_SUBCORES
LANES = 16
BATCH = 16384
K = 32
BPW = BATCH // NW
CHUNK = 128
NCH = BPW // CHUNK
GPC = CHUNK // LANES


def _mf_body(uid_hbm, iid_hbm, user_hbm, item_hbm, bu_hbm, bi_hbm, bias_hbm,
             out_hbm,
             uid_v, iid_v, ubuf0, ubuf1, ibuf0, ibuf1, bu_v, bi_v, bias_v,
             out_v, sem):
    cid = lax.axis_index("c")
    sid = lax.axis_index("s")
    wid = sid * N_CORES + cid
    base = wid * BPW

    pltpu.sync_copy(uid_hbm.at[pl.ds(base, BPW)], uid_v)
    pltpu.sync_copy(iid_hbm.at[pl.ds(base, BPW)], iid_v)
    pltpu.sync_copy(bu_hbm.at[pl.ds(base, BPW)], bu_v)
    pltpu.sync_copy(bi_hbm.at[pl.ds(base, BPW)], bi_v)
    pltpu.sync_copy(bias_hbm, bias_v)

    bvec = bias_v[...]
    iota16 = lax.iota(jnp.int32, LANES)

    ubufs = (ubuf0, ubuf1)
    ibufs = (ibuf0, ibuf1)

    def fire(j):
        ub = ubufs[j % 2]
        ib = ibufs[j % 2]

        def enqueue_group(g, carry):
            uvec = uid_v[pl.ds(j * CHUNK + g * LANES, LANES)]
            ivec = iid_v[pl.ds(j * CHUNK + g * LANES, LANES)]
            for l in range(LANES):
                d = g * LANES + l
                pltpu.async_copy(user_hbm.at[pl.ds(uvec[l], 1)],
                                 ub.at[pl.ds(d, 1)], sem)
                pltpu.async_copy(item_hbm.at[pl.ds(ivec[l], 1)],
                                 ib.at[pl.ds(d, 1)], sem)
            return carry

        lax.fori_loop(0, GPC, enqueue_group, 0)

    def drain(j):
        pltpu.make_async_copy(user_hbm.at[pl.ds(0, CHUNK)], ubufs[j % 2],
                              sem).wait()
        pltpu.make_async_copy(item_hbm.at[pl.ds(0, CHUNK)], ibufs[j % 2],
                              sem).wait()

    def compute(j):
        ub = ubufs[j % 2]
        ib = ibufs[j % 2]

        def group_body(g, carry):
            outv = jnp.zeros((LANES,), jnp.float32)
            for l in range(LANES):
                r = g * LANES + l
                u0 = ub[r, pl.ds(0, LANES)]
                u1 = ub[r, pl.ds(LANES, LANES)]
                i0 = ib[r, pl.ds(0, LANES)]
                i1 = ib[r, pl.ds(LANES, LANES)]
                prod = u0 * i0 + u1 * i1
                outv = jnp.where(iota16 == l, jnp.sum(prod), outv)
            o = j * CHUNK + g * LANES
            bu16 = bu_v[pl.ds(o, LANES)]
            bi16 = bi_v[pl.ds(o, LANES)]
            out_v[pl.ds(o, LANES)] = outv + bu16 + bi16 + bvec
            return carry

        lax.fori_loop(0, GPC, group_body, 0)

    fire(0)
    fire(1)
    for j in range(NCH):
        drain(j)
        compute(j)
        if j + 2 < NCH:
            fire(j + 2)

    pltpu.sync_copy(out_v, out_hbm.at[pl.ds(base, BPW)])


_mf = pl.kernel(
    _mf_body,
    mesh=plsc.VectorSubcoreMesh(core_axis_name="c", subcore_axis_name="s"),
    out_type=jax.ShapeDtypeStruct((BATCH,), jnp.float32),
    compiler_params=pltpu.CompilerParams(needs_layout_passes=False,
                                         use_tc_tiling_on_sc=True),
    scratch_types=[
        pltpu.VMEM((BPW,), jnp.int32),
        pltpu.VMEM((BPW,), jnp.int32),
        pltpu.VMEM((CHUNK, K), jnp.float32),
        pltpu.VMEM((CHUNK, K), jnp.float32),
        pltpu.VMEM((CHUNK, K), jnp.float32),
        pltpu.VMEM((CHUNK, K), jnp.float32),
        pltpu.VMEM((BPW,), jnp.float32),
        pltpu.VMEM((BPW,), jnp.float32),
        pltpu.VMEM((LANES,), jnp.float32),
        pltpu.VMEM((BPW,), jnp.float32),
        pltpu.SemaphoreType.DMA,
    ],
)


def kernel(train_x, user_w, item_w, bias_user_w, bias_item_w, bias):
    uid = train_x[:, 0]
    iid = train_x[:, 1]
    bu = jnp.take(bias_user_w.reshape(-1), uid, mode="clip")
    bi = jnp.take(bias_item_w.reshape(-1), iid, mode="clip")
    bias16 = jnp.broadcast_to(bias, (LANES,))
    return _mf(uid, iid, user_w, item_w, bu, bi, bias16)
